# baseline (speedup 1.0000x reference)
import functools

import jax
import jax.numpy as jnp
from jax import lax
from jax.experimental import pallas as pl
from jax.experimental.pallas import tpu as pltpu
from jax.experimental.pallas import tpu_sc as plsc

NE = 8
RK = 16
SCALE = 2.0
BF = jnp.bfloat16
F32 = jnp.float32
L = 16


def _logits_body(gw_ref, x_ref, o_ref):
    cw = o_ref.shape[2]
    for j in range(o_ref.shape[0]):
        xs = x_ref[pl.ds(j * cw, cw), :]
        o_ref[j] = lax.dot_general(gw_ref[...], xs, (((1,), (1,)), ((), ())),
                                   preferred_element_type=F32)


def _router_body(l_hbm, o_hbm, l_v, o_v):
    info = plsc.get_sparse_core_info()
    nc = info.num_cores
    wid = lax.axis_index("s") * nc + lax.axis_index("c")
    pltpu.sync_copy(l_hbm.at[wid], l_v)
    nvec = l_v.shape[1] // L
    for i in range(nvec):
        sl = pl.ds(i * L, L)
        le = [l_v[e, sl] for e in range(NE)]
        m1 = le[0]
        for e in range(1, NE):
            m1 = jnp.maximum(m1, le[e])
        e0 = jnp.full((L,), float(NE), F32)
        for e in range(NE - 1, -1, -1):
            e0 = jnp.where(le[e] >= m1, float(e), e0)
        m2 = jnp.full((L,), -1e30, F32)
        for e in range(NE):
            l2e = jnp.where(e0 == float(e), -1e30, le[e])
            m2 = jnp.maximum(m2, l2e)
        e1 = jnp.full((L,), float(NE), F32)
        for e in range(NE - 1, -1, -1):
            l2e = jnp.where(e0 == float(e), -1e30, le[e])
            e1 = jnp.where(l2e >= m2, float(e), e1)
        w0 = 1.0 / (1.0 + jnp.exp(m2 - m1))
        o_v[0, sl] = e0
        o_v[1, sl] = e1
        o_v[2, sl] = w0
        o_v[3, sl] = 1.0 - w0
    pltpu.sync_copy(o_v, o_hbm.at[wid])


_DNT = (((1,), (1,)), ((), ()))


def _moe_body(x_ref, ew_ref, wg_ref, wu_ref, wd_ref, ag_ref, au_ref,
              bg_ref, bu_ref, ad_ref, bd_ref, o_ref):
    x32 = x_ref[...]
    xb = x32.astype(BF)
    TT = xb.shape[0]

    nw = ew_ref.shape[0]
    ew = jnp.concatenate([ew_ref[c] for c in range(nw)], axis=-1)
    ewt = ew.T
    e0 = ewt[:, 0:1].astype(jnp.int32)
    e1 = ewt[:, 1:2].astype(jnp.int32)
    w0 = ewt[:, 2:3]
    w1 = ewt[:, 3:4]

    ce = lax.broadcasted_iota(jnp.int32, (TT, NE * RK), 1) // RK
    mk0 = (ce == e0).astype(BF)
    mk1 = (ce == e1).astype(BF)

    cg = lax.dot_general(xb, wg_ref[...], _DNT,
                         preferred_element_type=F32)
    cu = lax.dot_general(xb, wu_ref[...], _DNT,
                         preferred_element_type=F32)
    ug = lax.dot_general(xb, ag_ref[...], _DNT,
                         preferred_element_type=F32).astype(BF)
    uu = lax.dot_general(xb, au_ref[...], _DNT,
                         preferred_element_type=F32).astype(BF)

    ugm = jnp.concatenate([ug * mk0, ug * mk1], axis=0)
    uum = jnp.concatenate([uu * mk0, uu * mk1], axis=0)
    dg = jnp.dot(ugm, bg_ref[...], preferred_element_type=F32)
    du = jnp.dot(uum, bu_ref[...], preferred_element_type=F32)

    def slot(k, mk, w):
        g = (cg + SCALE * dg[k * TT:(k + 1) * TT]).astype(BF)
        u = (cu + SCALE * du[k * TT:(k + 1) * TT]).astype(BF)
        z = (g * jax.nn.sigmoid(g)) * u * w.astype(BF)
        v = lax.dot_general(z, ad_ref[...], _DNT,
                            preferred_element_type=F32)
        return z, (v * mk.astype(F32)).astype(BF)

    z0, v0 = slot(0, mk0, w0)
    z1, v1 = slot(1, mk1, w1)
    out = lax.dot_general(z0 + z1, wd_ref[...], _DNT,
                          preferred_element_type=F32)
    out = out + SCALE * jnp.dot(v0 + v1, bd_ref[...],
                                preferred_element_type=F32)
    o_ref[...] = out


@jax.jit
def _run(x, gate_w, Wg16, Wu16, Wd16, Ag2, Au2, BgT, BuT, Ad2, BdT):
    T, D = x.shape
    FF = Wg16.shape[0]
    TT = 256
    NW = 32
    CW = T // NW

    TB = 1024
    logitsT = pl.pallas_call(
        _logits_body,
        grid=(T // TB,),
        in_specs=[
            pl.BlockSpec((NE, D), lambda i: (0, 0)),
            pl.BlockSpec((TB, D), lambda i: (i, 0)),
        ],
        out_specs=pl.BlockSpec((TB // CW, NE, CW), lambda i: (i, 0, 0)),
        out_shape=jax.ShapeDtypeStruct((NW, NE, CW), F32),
    )(gate_w, x)

    mesh = plsc.VectorSubcoreMesh(core_axis_name="c", subcore_axis_name="s")
    ew = pl.kernel(
        _router_body,
        mesh=mesh,
        out_type=jax.ShapeDtypeStruct((NW, 4, CW), F32),
        scratch_types=[
            pltpu.VMEM((NE, CW), F32),
            pltpu.VMEM((4, CW), F32),
        ],
    )(logitsT)

    const = lambda shape: pl.BlockSpec(shape, lambda i: (0, 0))
    return pl.pallas_call(
        _moe_body,
        grid=(T // TT,),
        in_specs=[
            pl.BlockSpec((TT, D), lambda i: (i, 0)),
            pl.BlockSpec((TT // CW, 4, CW), lambda i: (i, 0, 0)),
            const((FF, D)), const((FF, D)), const((D, FF)),
            const((NE * RK, D)), const((NE * RK, D)),
            const((NE * RK, FF)), const((NE * RK, FF)),
            const((NE * RK, FF)), const((NE * RK, D)),
        ],
        out_specs=pl.BlockSpec((TT, D), lambda i: (i, 0)),
        out_shape=jax.ShapeDtypeStruct((T, D), F32),
    )(x, ew, Wg16, Wu16, Wd16, Ag2, Au2, BgT, BuT, Ad2, BdT)


def kernel(hidden_states, gate_w, Wg, Wu, Wd, Ag, Bg, Au, Bu, Ad, Bd):
    B, S, D = hidden_states.shape
    x = hidden_states.reshape(B * S, D)
    Wg16 = Wg.astype(BF)
    Wu16 = Wu.astype(BF)
    Wd16 = Wd.astype(BF)
    Ag2 = Ag.reshape(NE * RK, -1).astype(BF)
    Au2 = Au.reshape(NE * RK, -1).astype(BF)
    Ad2 = Ad.reshape(NE * RK, -1).astype(BF)
    BgT = Bg.transpose(0, 2, 1).reshape(NE * RK, -1).astype(BF)
    BuT = Bu.transpose(0, 2, 1).reshape(NE * RK, -1).astype(BF)
    BdT = Bd.transpose(0, 2, 1).reshape(NE * RK, -1).astype(BF)
    out = _run(x, gate_w, Wg16, Wu16, Wd16, Ag2, Au2, BgT, BuT, Ad2, BdT)
    return out.reshape(B, S, D)


# speedup vs baseline: 1.0050x; 1.0050x over previous
import functools

import jax
import jax.numpy as jnp
from jax import lax
from jax.experimental import pallas as pl
from jax.experimental.pallas import tpu as pltpu
from jax.experimental.pallas import tpu_sc as plsc

NE = 8
RK = 16
SCALE = 2.0
BF = jnp.bfloat16
F32 = jnp.float32
L = 16


def _logits_body(gw_ref, x_ref, o_ref):
    cw = o_ref.shape[2]
    for j in range(o_ref.shape[0]):
        xs = x_ref[pl.ds(j * cw, cw), :]
        o_ref[j] = lax.dot_general(gw_ref[...], xs, (((1,), (1,)), ((), ())),
                                   preferred_element_type=F32)


def _router_body(l_hbm, o_hbm, l_v, o_v):
    info = plsc.get_sparse_core_info()
    nc = info.num_cores
    wid = lax.axis_index("s") * nc + lax.axis_index("c")
    pltpu.sync_copy(l_hbm.at[wid], l_v)
    nvec = l_v.shape[1] // L
    for i in range(nvec):
        sl = pl.ds(i * L, L)
        le = [l_v[e, sl] for e in range(NE)]
        m1 = le[0]
        for e in range(1, NE):
            m1 = jnp.maximum(m1, le[e])
        e0 = jnp.full((L,), float(NE), F32)
        for e in range(NE - 1, -1, -1):
            e0 = jnp.where(le[e] >= m1, float(e), e0)
        m2 = jnp.full((L,), -1e30, F32)
        for e in range(NE):
            l2e = jnp.where(e0 == float(e), -1e30, le[e])
            m2 = jnp.maximum(m2, l2e)
        e1 = jnp.full((L,), float(NE), F32)
        for e in range(NE - 1, -1, -1):
            l2e = jnp.where(e0 == float(e), -1e30, le[e])
            e1 = jnp.where(l2e >= m2, float(e), e1)
        w0 = 1.0 / (1.0 + jnp.exp(m2 - m1))
        o_v[0, sl] = e0
        o_v[1, sl] = e1
        o_v[2, sl] = w0
        o_v[3, sl] = 1.0 - w0
    pltpu.sync_copy(o_v, o_hbm.at[wid])


_DNT = (((1,), (1,)), ((), ()))


def _moe_body(x_ref, ew_ref, wg_ref, wu_ref, wd_ref, ag_ref, au_ref,
              bg_ref, bu_ref, ad_ref, bd_ref, o_ref):
    x32 = x_ref[...]
    xb = x32.astype(BF)
    TT = xb.shape[0]

    nw = ew_ref.shape[0]
    ew = jnp.concatenate([ew_ref[c] for c in range(nw)], axis=-1)
    ewt = ew.T
    e0 = ewt[:, 0:1].astype(jnp.int32)
    e1 = ewt[:, 1:2].astype(jnp.int32)
    w0 = ewt[:, 2:3]
    w1 = ewt[:, 3:4]

    ce = lax.broadcasted_iota(jnp.int32, (TT, NE * RK), 1) // RK
    mk0 = (ce == e0).astype(BF)
    mk1 = (ce == e1).astype(BF)

    cg = lax.dot_general(xb, wg_ref[...], _DNT,
                         preferred_element_type=F32)
    cu = lax.dot_general(xb, wu_ref[...], _DNT,
                         preferred_element_type=F32)
    ug = lax.dot_general(xb, ag_ref[...], _DNT,
                         preferred_element_type=F32).astype(BF)
    uu = lax.dot_general(xb, au_ref[...], _DNT,
                         preferred_element_type=F32).astype(BF)

    ugm = jnp.concatenate([ug * mk0, ug * mk1], axis=0)
    uum = jnp.concatenate([uu * mk0, uu * mk1], axis=0)
    dg = jnp.dot(ugm, bg_ref[...], preferred_element_type=F32)
    du = jnp.dot(uum, bu_ref[...], preferred_element_type=F32)

    def slot(k, mk, w):
        g = (cg + SCALE * dg[k * TT:(k + 1) * TT]).astype(BF)
        u = (cu + SCALE * du[k * TT:(k + 1) * TT]).astype(BF)
        z = (g * jax.nn.sigmoid(g)) * u * w.astype(BF)
        v = lax.dot_general(z, ad_ref[...], _DNT,
                            preferred_element_type=F32)
        return z, (v * mk.astype(F32)).astype(BF)

    z0, v0 = slot(0, mk0, w0)
    z1, v1 = slot(1, mk1, w1)
    out = lax.dot_general(z0 + z1, wd_ref[...], _DNT,
                          preferred_element_type=F32)
    out = out + SCALE * jnp.dot(v0 + v1, bd_ref[...],
                                preferred_element_type=F32)
    o_ref[...] = out


@jax.jit
def _run(x, gate_w, Wg, Wu, Wd, Ag2f, Au2f, BgTf, BuTf, Ad2f, BdTf):
    T, D = x.shape
    FF = Wg.shape[0]
    TT = 256
    NW = 32
    CW = T // NW

    TB = 1024
    logitsT = pl.pallas_call(
        _logits_body,
        grid=(T // TB,),
        in_specs=[
            pl.BlockSpec((NE, D), lambda i: (0, 0)),
            pl.BlockSpec((TB, D), lambda i: (i, 0)),
        ],
        out_specs=pl.BlockSpec((TB // CW, NE, CW), lambda i: (i, 0, 0)),
        out_shape=jax.ShapeDtypeStruct((NW, NE, CW), F32),
    )(gate_w, x)

    mesh = plsc.VectorSubcoreMesh(core_axis_name="c", subcore_axis_name="s")
    ew = pl.kernel(
        _router_body,
        mesh=mesh,
        out_type=jax.ShapeDtypeStruct((NW, 4, CW), F32),
        scratch_types=[
            pltpu.VMEM((NE, CW), F32),
            pltpu.VMEM((4, CW), F32),
        ],
    )(logitsT)

    tie = 0.0 * logitsT[0, 0, 0]
    cast = lambda a: (a + tie).astype(BF)
    Wg16, Wu16, Wd16 = cast(Wg), cast(Wu), cast(Wd)
    Ag2, Au2, Ad2 = cast(Ag2f), cast(Au2f), cast(Ad2f)
    BgT, BuT, BdT = cast(BgTf), cast(BuTf), cast(BdTf)

    const = lambda shape: pl.BlockSpec(shape, lambda i: (0, 0))
    return pl.pallas_call(
        _moe_body,
        grid=(T // TT,),
        in_specs=[
            pl.BlockSpec((TT, D), lambda i: (i, 0)),
            pl.BlockSpec((TT // CW, 4, CW), lambda i: (i, 0, 0)),
            const((FF, D)), const((FF, D)), const((D, FF)),
            const((NE * RK, D)), const((NE * RK, D)),
            const((NE * RK, FF)), const((NE * RK, FF)),
            const((NE * RK, FF)), const((NE * RK, D)),
        ],
        out_specs=pl.BlockSpec((TT, D), lambda i: (i, 0)),
        out_shape=jax.ShapeDtypeStruct((T, D), F32),
    )(x, ew, Wg16, Wu16, Wd16, Ag2, Au2, BgT, BuT, Ad2, BdT)


def kernel(hidden_states, gate_w, Wg, Wu, Wd, Ag, Bg, Au, Bu, Ad, Bd):
    B, S, D = hidden_states.shape
    x = hidden_states.reshape(B * S, D)
    Ag2f = Ag.reshape(NE * RK, -1)
    Au2f = Au.reshape(NE * RK, -1)
    Ad2f = Ad.reshape(NE * RK, -1)
    BgTf = Bg.transpose(0, 2, 1).reshape(NE * RK, -1)
    BuTf = Bu.transpose(0, 2, 1).reshape(NE * RK, -1)
    BdTf = Bd.transpose(0, 2, 1).reshape(NE * RK, -1)
    out = _run(x, gate_w, Wg, Wu, Wd, Ag2f, Au2f, BgTf, BuTf, Ad2f, BdTf)
    return out.reshape(B, S, D)


# speedup vs baseline: 1.0068x; 1.0018x over previous
import jax
import jax.numpy as jnp
from jax import lax
from jax.experimental import pallas as pl
from jax.experimental.pallas import tpu as pltpu
from jax.experimental.pallas import tpu_sc as plsc

NE = 8
RK = 16
SCALE = 2.0
BF = jnp.bfloat16
F32 = jnp.float32
L = 16


def _logits_body(gw_ref, x_ref, o_ref):
    cw = o_ref.shape[2]
    for j in range(o_ref.shape[0]):
        xs = x_ref[pl.ds(j * cw, cw), :]
        o_ref[j] = lax.dot_general(gw_ref[...], xs, (((1,), (1,)), ((), ())),
                                   preferred_element_type=F32)


def _router_body(l_hbm, o_hbm, l_v, o_v):
    info = plsc.get_sparse_core_info()
    nc = info.num_cores
    wid = lax.axis_index("s") * nc + lax.axis_index("c")
    pltpu.sync_copy(l_hbm.at[wid], l_v)
    nvec = l_v.shape[1] // L
    for i in range(nvec):
        sl = pl.ds(i * L, L)
        le = [l_v[e, sl] for e in range(NE)]
        m1 = le[0]
        for e in range(1, NE):
            m1 = jnp.maximum(m1, le[e])
        e0 = jnp.full((L,), float(NE), F32)
        for e in range(NE - 1, -1, -1):
            e0 = jnp.where(le[e] >= m1, float(e), e0)
        m2 = jnp.full((L,), -1e30, F32)
        for e in range(NE):
            l2e = jnp.where(e0 == float(e), -1e30, le[e])
            m2 = jnp.maximum(m2, l2e)
        e1 = jnp.full((L,), float(NE), F32)
        for e in range(NE - 1, -1, -1):
            l2e = jnp.where(e0 == float(e), -1e30, le[e])
            e1 = jnp.where(l2e >= m2, float(e), e1)
        w0 = 1.0 / (1.0 + jnp.exp(m2 - m1))
        o_v[0, sl] = e0
        o_v[1, sl] = e1
        o_v[2, sl] = w0
        o_v[3, sl] = 1.0 - w0
    pltpu.sync_copy(o_v, o_hbm.at[wid])


_DNT = (((1,), (1,)), ((), ()))


def _moe_body(x_ref, ew_ref, wg_ref, wu_ref, wd_ref, ag_ref, au_ref,
              bg_ref, bu_ref, ad_ref, bd_ref, o_ref):
    x32 = x_ref[...]
    xb = x32.astype(BF)
    TT = xb.shape[0]

    nw = ew_ref.shape[0]
    ew = jnp.concatenate([ew_ref[c] for c in range(nw)], axis=-1)
    ewt = ew.T
    e0 = ewt[:, 0:1].astype(jnp.int32)
    e1 = ewt[:, 1:2].astype(jnp.int32)
    w0 = ewt[:, 2:3]
    w1 = ewt[:, 3:4]

    ce = lax.broadcasted_iota(jnp.int32, (TT, NE * RK), 1) // RK
    mk0 = (ce == e0).astype(BF)
    mk1 = (ce == e1).astype(BF)

    cg = lax.dot_general(xb, wg_ref[...], _DNT,
                         preferred_element_type=F32)
    cu = lax.dot_general(xb, wu_ref[...], _DNT,
                         preferred_element_type=F32)
    ug = lax.dot_general(xb, ag_ref[...], _DNT,
                         preferred_element_type=F32).astype(BF)
    uu = lax.dot_general(xb, au_ref[...], _DNT,
                         preferred_element_type=F32).astype(BF)

    ugm = jnp.concatenate([ug * mk0, ug * mk1], axis=0)
    uum = jnp.concatenate([uu * mk0, uu * mk1], axis=0)
    dg = jnp.dot(ugm, bg_ref[...], preferred_element_type=F32)
    du = jnp.dot(uum, bu_ref[...], preferred_element_type=F32)

    def slot(k, mk, w):
        g = (cg + SCALE * dg[k * TT:(k + 1) * TT]).astype(BF)
        u = (cu + SCALE * du[k * TT:(k + 1) * TT]).astype(BF)
        z = (g * jax.nn.sigmoid(g)) * u * w.astype(BF)
        v = lax.dot_general(z, ad_ref[...], _DNT,
                            preferred_element_type=F32)
        return z, (v * mk.astype(F32)).astype(BF)

    z0, v0 = slot(0, mk0, w0)
    z1, v1 = slot(1, mk1, w1)
    out = lax.dot_general(z0 + z1, wd_ref[...], _DNT,
                          preferred_element_type=F32)
    out = out + SCALE * jnp.dot(v0 + v1, bd_ref[...],
                                preferred_element_type=F32)
    o_ref[...] = out


@jax.jit
def _run(x, gate_w, Wg, Wu, Wd, Ag2f, Au2f, BgTf, BuTf, Ad2f, BdTf):
    T, D = x.shape
    FF = Wg.shape[0]
    TT = 256
    NW = 32
    CW = T // NW

    TB = 2048
    logitsT = pl.pallas_call(
        _logits_body,
        grid=(T // TB,),
        in_specs=[
            pl.BlockSpec((NE, D), lambda i: (0, 0)),
            pl.BlockSpec((TB, D), lambda i: (i, 0)),
        ],
        out_specs=pl.BlockSpec((TB // CW, NE, CW), lambda i: (i, 0, 0)),
        out_shape=jax.ShapeDtypeStruct((NW, NE, CW), F32),
    )(gate_w, x)

    mesh = plsc.VectorSubcoreMesh(core_axis_name="c", subcore_axis_name="s")
    ew = pl.kernel(
        _router_body,
        mesh=mesh,
        out_type=jax.ShapeDtypeStruct((NW, 4, CW), F32),
        scratch_types=[
            pltpu.VMEM((NE, CW), F32),
            pltpu.VMEM((4, CW), F32),
        ],
    )(logitsT)

    tie = 0.0 * logitsT[0, 0, 0]
    cast = lambda a: (a + tie).astype(BF)
    Wg16, Wu16, Wd16 = cast(Wg), cast(Wu), cast(Wd)
    Ag2, Au2, Ad2 = cast(Ag2f), cast(Au2f), cast(Ad2f)
    BgT, BuT, BdT = cast(BgTf), cast(BuTf), cast(BdTf)

    const = lambda shape: pl.BlockSpec(shape, lambda i: (0, 0))
    return pl.pallas_call(
        _moe_body,
        grid=(T // TT,),
        in_specs=[
            pl.BlockSpec((TT, D), lambda i: (i, 0)),
            pl.BlockSpec((TT // CW, 4, CW), lambda i: (i, 0, 0)),
            const((FF, D)), const((FF, D)), const((D, FF)),
            const((NE * RK, D)), const((NE * RK, D)),
            const((NE * RK, FF)), const((NE * RK, FF)),
            const((NE * RK, FF)), const((NE * RK, D)),
        ],
        out_specs=pl.BlockSpec((TT, D), lambda i: (i, 0)),
        out_shape=jax.ShapeDtypeStruct((T, D), F32),
    )(x, ew, Wg16, Wu16, Wd16, Ag2, Au2, BgT, BuT, Ad2, BdT)


def kernel(hidden_states, gate_w, Wg, Wu, Wd, Ag, Bg, Au, Bu, Ad, Bd):
    B, S, D = hidden_states.shape
    x = hidden_states.reshape(B * S, D)
    Ag2f = Ag.reshape(NE * RK, -1)
    Au2f = Au.reshape(NE * RK, -1)
    Ad2f = Ad.reshape(NE * RK, -1)
    BgTf = Bg.transpose(0, 2, 1).reshape(NE * RK, -1)
    BuTf = Bu.transpose(0, 2, 1).reshape(NE * RK, -1)
    BdTf = Bd.transpose(0, 2, 1).reshape(NE * RK, -1)
    out = _run(x, gate_w, Wg, Wu, Wd, Ag2f, Au2f, BgTf, BuTf, Ad2f, BdTf)
    return out.reshape(B, S, D)
